# baseline (speedup 1.0000x reference)
import functools
import math

import jax
import jax.numpy as jnp
from jax import lax
from jax.experimental import pallas as pl
from jax.experimental.pallas import tpu as pltpu
from jax.experimental.pallas import tpu_sc as plsc

E = 8
HID = 1024
INTER = 2048
NT = 4
TILE_I = INTER // NT
NW = 32


def _dispatch(x_flat, flat1, flat2, n_rows):
    T = x_flat.shape[0]
    tpw = T // NW
    nchunk = tpw // 64

    @functools.partial(
        pl.kernel,
        out_type=jax.ShapeDtypeStruct((n_rows, HID), jnp.float32),
        mesh=plsc.VectorSubcoreMesh(core_axis_name="c", subcore_axis_name="s"),
        scratch_types=[
            pltpu.VMEM((64,), jnp.int32),
            pltpu.VMEM((64,), jnp.int32),
            pltpu.VMEM((64, HID), jnp.float32),
            pltpu.SemaphoreType.DMA,
        ],
    )
    def k(x_hbm, f1_hbm, f2_hbm, X_hbm, idx1_v, idx2_v, rows_v, sem):
        wid = lax.axis_index("s") * 2 + lax.axis_index("c")
        base = wid * tpw

        def body(i, carry):
            off = base + i * 64
            pltpu.sync_copy(f1_hbm.at[pl.ds(off, 64)], idx1_v)
            pltpu.sync_copy(f2_hbm.at[pl.ds(off, 64)], idx2_v)
            pltpu.sync_copy(x_hbm.at[pl.ds(off, 64)], rows_v)
            c1 = pltpu.async_copy(rows_v, X_hbm.at[idx1_v], sem)
            c2 = pltpu.async_copy(rows_v, X_hbm.at[idx2_v], sem)
            c1.wait()
            c2.wait()
            return carry

        lax.fori_loop(0, nchunk, body, 0)

    return k(x_flat, flat1, flat2)


def _combine(Yw, flat1, flat2, T):
    tpw = T // NW
    CH = 32
    nchunk = tpw // CH

    @functools.partial(
        pl.kernel,
        out_type=jax.ShapeDtypeStruct((T, HID), jnp.float32),
        mesh=plsc.VectorSubcoreMesh(core_axis_name="c", subcore_axis_name="s"),
        scratch_types=[
            pltpu.VMEM((CH,), jnp.int32),
            pltpu.VMEM((CH,), jnp.int32),
            pltpu.VMEM((CH, HID), jnp.float32),
            pltpu.VMEM((CH, HID), jnp.float32),
            pltpu.SemaphoreType.DMA,
        ],
    )
    def k(y_hbm, f1_hbm, f2_hbm, out_hbm, idx1_v, idx2_v, rows1_v, rows2_v, sem):
        wid = lax.axis_index("s") * 2 + lax.axis_index("c")
        base = wid * tpw

        def body(i, carry):
            off = base + i * CH
            pltpu.sync_copy(f1_hbm.at[pl.ds(off, CH)], idx1_v)
            pltpu.sync_copy(f2_hbm.at[pl.ds(off, CH)], idx2_v)
            g1 = pltpu.async_copy(y_hbm.at[idx1_v], rows1_v, sem)
            g2 = pltpu.async_copy(y_hbm.at[idx2_v], rows2_v, sem)
            g1.wait()
            g2.wait()

            def row_body(r, c2):
                def col_body(j, c3):
                    sl = pl.ds(j * 16, 16)
                    rows1_v[r, sl] = rows1_v[r, sl] + rows2_v[r, sl]
                    return c3

                return lax.fori_loop(0, HID // 16, col_body, c2)

            lax.fori_loop(0, CH, row_body, 0)
            pltpu.sync_copy(rows1_v, out_hbm.at[pl.ds(off, CH)])
            return carry

        lax.fori_loop(0, nchunk, body, 0)

    return k(Yw, flat1, flat2)


def _ffn_body(x_ref, w1_ref, w2_ref, dw_ref, y_ref, acc_ref):
    e = pl.program_id(0)
    nt = pl.program_id(1)

    @pl.when(e < E)
    def _():
        h = jnp.dot(x_ref[...], w1_ref[0], preferred_element_type=jnp.float32)
        h = jnp.maximum(h, 0.0)
        part = jnp.dot(h, w2_ref[0], preferred_element_type=jnp.float32)

        @pl.when(nt == 0)
        def _():
            acc_ref[...] = part

        @pl.when(nt > 0)
        def _():
            acc_ref[...] = acc_ref[...] + part

        @pl.when(nt == NT - 1)
        def _():
            dw = dw_ref[...]
            y_ref[...] = jnp.where(dw > 0.0, acc_ref[...] * dw, 0.0)

    @pl.when((e == E) & (nt == NT - 1))
    def _():
        y_ref[...] = jnp.zeros_like(y_ref)


def _ffn(x_disp, experts_inter, experts_out, dw, cap, n_rows):
    return pl.pallas_call(
        _ffn_body,
        grid=(E + 1, NT),
        in_specs=[
            pl.BlockSpec((cap, HID), lambda e, n: (e, 0)),
            pl.BlockSpec((1, HID, TILE_I), lambda e, n: (jnp.minimum(e, E - 1), 0, n)),
            pl.BlockSpec((1, TILE_I, HID), lambda e, n: (jnp.minimum(e, E - 1), n, 0)),
            pl.BlockSpec((cap, 1), lambda e, n: (e, 0)),
        ],
        out_specs=pl.BlockSpec((cap, HID), lambda e, n: (e, 0)),
        out_shape=jax.ShapeDtypeStruct((n_rows, HID), jnp.float32),
        scratch_shapes=[pltpu.VMEM((cap, HID), jnp.float32)],
        compiler_params=pltpu.CompilerParams(
            dimension_semantics=("arbitrary", "arbitrary"),
        ),
    )(x_disp, experts_inter, experts_out, dw)


def kernel(x, experts_inter, experts_out, router_w, router_b):
    b, s, hid = x.shape
    T = b * s
    cap = math.ceil(T / E * 1.0)
    n_rows = (E + 1) * cap
    dump = E * cap
    x_flat = x.reshape(T, hid)

    logits = x_flat @ router_w.T + router_b
    probs = jax.nn.softmax(logits, axis=-1)
    rows = jnp.arange(T)
    i1 = jnp.argmax(probs, axis=-1)
    v1 = jnp.take_along_axis(probs, i1[:, None], axis=-1)[:, 0]
    masked = probs.at[rows, i1].set(-jnp.inf)
    i2 = jnp.argmax(masked, axis=-1)
    v2 = jnp.take_along_axis(probs, i2[:, None], axis=-1)[:, 0]

    mask = jnp.zeros((T, E), jnp.int32).at[rows, i1].set((v1 > 0).astype(jnp.int32))
    mask = mask.at[rows, i2].set((v2 > 0).astype(jnp.int32))
    slots = jnp.cumsum(mask, axis=0) - mask
    c1 = jnp.take_along_axis(slots, i1[:, None], axis=-1)[:, 0]
    c2 = jnp.take_along_axis(slots, i2[:, None], axis=-1)[:, 0]
    ok1 = (c1 < cap) & (v1 > 0)
    ok2 = (c2 < cap) & (v2 > 0)
    flat1 = jnp.where(ok1, i1 * cap + c1, dump).astype(jnp.int32)
    flat2 = jnp.where(ok2, i2 * cap + c2, dump).astype(jnp.int32)

    dw = jnp.zeros((n_rows,), jnp.float32).at[flat1].add(jnp.where(ok1, v1, 0.0))
    dw = dw.at[flat2].add(jnp.where(ok2, v2, 0.0))

    X = _dispatch(x_flat, flat1, flat2, n_rows)
    Yw = _ffn(X, experts_inter, experts_out, dw[:, None], cap, n_rows)
    out = _combine(Yw, flat1, flat2, T)
    return out.reshape(b, s, hid)


# speedup vs baseline: 1.0107x; 1.0107x over previous
import functools
import math

import jax
import jax.numpy as jnp
from jax import lax
from jax.experimental import pallas as pl
from jax.experimental.pallas import tpu as pltpu
from jax.experimental.pallas import tpu_sc as plsc

E = 8
HID = 1024
INTER = 2048
NT = 4
TILE_I = INTER // NT
NW = 32


DCH = 32


def _dispatch(x_flat, flatd, n_rows):
    T = x_flat.shape[0]
    tpw = T // NW
    nchunk = tpw // DCH

    @functools.partial(
        pl.kernel,
        out_type=jax.ShapeDtypeStruct((n_rows, HID), jnp.float32),
        mesh=plsc.VectorSubcoreMesh(core_axis_name="c", subcore_axis_name="s"),
        scratch_types=[
            pltpu.VMEM((nchunk, 2, DCH), jnp.int32),
            pltpu.VMEM((DCH, HID), jnp.float32),
            pltpu.VMEM((DCH, HID), jnp.float32),
            pltpu.SemaphoreType.DMA,
            pltpu.SemaphoreType.DMA,
            pltpu.SemaphoreType.DMA,
            pltpu.SemaphoreType.DMA,
        ],
    )
    def k(x_hbm, fd_hbm, X_hbm, idx_v, rows0, rows1, si0, si1, ss0, ss1):
        wid = lax.axis_index("s") * 2 + lax.axis_index("c")
        base = wid * tpw
        bufs = (rows0, rows1)
        isems = (si0, si1)
        ssems = (ss0, ss1)
        pltpu.sync_copy(fd_hbm.at[wid], idx_v)

        def issue_in(j, p):
            @pl.when(j < nchunk)
            def _():
                pltpu.async_copy(
                    x_hbm.at[pl.ds(base + j * DCH, DCH)], bufs[p], isems[p])

        def wait_in(j, p):
            pltpu.make_async_copy(
                x_hbm.at[pl.ds(base + j * DCH, DCH)], bufs[p], isems[p]).wait()

        def scat(j, p):
            pltpu.async_copy(bufs[p], X_hbm.at[idx_v.at[j, 0]], ssems[p])
            pltpu.async_copy(bufs[p], X_hbm.at[idx_v.at[j, 1]], ssems[p])
            pltpu.make_async_copy(bufs[p], X_hbm.at[idx_v.at[j, 0]], ssems[p]).wait()
            pltpu.make_async_copy(bufs[p], X_hbm.at[idx_v.at[j, 1]], ssems[p]).wait()

        issue_in(0, 0)

        def pair(pi, carry):
            i = 2 * pi
            issue_in(i + 1, 1)
            wait_in(i, 0)
            scat(i, 0)
            issue_in(i + 2, 0)
            wait_in(i + 1, 1)
            scat(i + 1, 1)
            return carry

        lax.fori_loop(0, nchunk // 2, pair, 0)

    return k(x_flat, flatd)


CCH = 16


def _combine(Yw, flatc, T):
    tpw = T // NW
    nchunk = tpw // CCH

    @functools.partial(
        pl.kernel,
        out_type=jax.ShapeDtypeStruct((T, HID), jnp.float32),
        mesh=plsc.VectorSubcoreMesh(core_axis_name="c", subcore_axis_name="s"),
        scratch_types=[
            pltpu.VMEM((nchunk * 2 * CCH,), jnp.int32),
            pltpu.VMEM((2 * CCH, HID), jnp.float32),
            pltpu.VMEM((2 * CCH, HID), jnp.float32),
            pltpu.SemaphoreType.DMA,
            pltpu.SemaphoreType.DMA,
            pltpu.SemaphoreType.DMA,
            pltpu.SemaphoreType.DMA,
        ],
    )
    def k(y_hbm, fc_hbm, out_hbm, idx_v, rows0, rows1, sg0, sg1, so0, so1):
        wid = lax.axis_index("s") * 2 + lax.axis_index("c")
        base = wid * tpw
        bufs = (rows0, rows1)
        gsems = (sg0, sg1)
        osems = (so0, so1)
        pltpu.sync_copy(fc_hbm.at[wid], idx_v)

        def issue_gather(j, p):
            @pl.when(j < nchunk)
            def _():
                @pl.when(j >= 2)
                def _():
                    pltpu.make_async_copy(
                        bufs[p].at[pl.ds(0, CCH)],
                        out_hbm.at[pl.ds(base + (j - 2) * CCH, CCH)],
                        osems[p]).wait()

                pltpu.async_copy(
                    y_hbm.at[idx_v.at[pl.ds(j * 2 * CCH, 2 * CCH)]],
                    bufs[p], gsems[p])

        def wait_gather(j, p):
            pltpu.make_async_copy(
                y_hbm.at[idx_v.at[pl.ds(j * 2 * CCH, 2 * CCH)]],
                bufs[p], gsems[p]).wait()

        def process(j, p):
            buf = bufs[p]

            def row_body(r, c):
                for jj in range(HID // 16):
                    sl = pl.ds(jj * 16, 16)
                    buf[r, sl] = buf[r, sl] + buf[CCH + r, sl]
                return c

            lax.fori_loop(0, CCH, row_body, 0)
            pltpu.async_copy(
                buf.at[pl.ds(0, CCH)],
                out_hbm.at[pl.ds(base + j * CCH, CCH)], osems[p])

        issue_gather(0, 0)

        def pair(pi, carry):
            i = 2 * pi
            issue_gather(i + 1, 1)
            wait_gather(i, 0)
            process(i, 0)
            issue_gather(i + 2, 0)
            wait_gather(i + 1, 1)
            process(i + 1, 1)
            return carry

        lax.fori_loop(0, nchunk // 2, pair, 0)
        pltpu.make_async_copy(
            bufs[0].at[pl.ds(0, CCH)],
            out_hbm.at[pl.ds(base + (nchunk - 2) * CCH, CCH)], osems[0]).wait()
        pltpu.make_async_copy(
            bufs[1].at[pl.ds(0, CCH)],
            out_hbm.at[pl.ds(base + (nchunk - 1) * CCH, CCH)], osems[1]).wait()

    return k(Yw, flatc)


def _ffn_body(x_ref, w1_ref, w2_ref, dw_ref, y_ref, acc_ref):
    e = pl.program_id(0)
    nt = pl.program_id(1)

    @pl.when(e < E)
    def _():
        h = jnp.dot(x_ref[...], w1_ref[0], preferred_element_type=jnp.float32)
        h = jnp.maximum(h, 0.0)
        part = jnp.dot(h, w2_ref[0], preferred_element_type=jnp.float32)

        @pl.when(nt == 0)
        def _():
            acc_ref[...] = part

        @pl.when(nt > 0)
        def _():
            acc_ref[...] = acc_ref[...] + part

        @pl.when(nt == NT - 1)
        def _():
            dw = dw_ref[...]
            y_ref[...] = jnp.where(dw > 0.0, acc_ref[...] * dw, 0.0)

    @pl.when((e == E) & (nt == NT - 1))
    def _():
        y_ref[...] = jnp.zeros_like(y_ref)


def _ffn(x_disp, experts_inter, experts_out, dw, cap, n_rows):
    return pl.pallas_call(
        _ffn_body,
        grid=(E + 1, NT),
        in_specs=[
            pl.BlockSpec((cap, HID), lambda e, n: (e, 0)),
            pl.BlockSpec((1, HID, TILE_I), lambda e, n: (jnp.minimum(e, E - 1), 0, n)),
            pl.BlockSpec((1, TILE_I, HID), lambda e, n: (jnp.minimum(e, E - 1), n, 0)),
            pl.BlockSpec((cap, 1), lambda e, n: (e, 0)),
        ],
        out_specs=pl.BlockSpec((cap, HID), lambda e, n: (e, 0)),
        out_shape=jax.ShapeDtypeStruct((n_rows, HID), jnp.float32),
        scratch_shapes=[pltpu.VMEM((cap, HID), jnp.float32)],
        compiler_params=pltpu.CompilerParams(
            dimension_semantics=("arbitrary", "arbitrary"),
        ),
    )(x_disp, experts_inter, experts_out, dw)


def kernel(x, experts_inter, experts_out, router_w, router_b):
    b, s, hid = x.shape
    T = b * s
    cap = math.ceil(T / E * 1.0)
    n_rows = (E + 1) * cap
    dump = E * cap
    x_flat = x.reshape(T, hid)

    logits = x_flat @ router_w.T + router_b
    probs = jax.nn.softmax(logits, axis=-1)
    rows = jnp.arange(T)
    i1 = jnp.argmax(probs, axis=-1)
    v1 = jnp.take_along_axis(probs, i1[:, None], axis=-1)[:, 0]
    masked = probs.at[rows, i1].set(-jnp.inf)
    i2 = jnp.argmax(masked, axis=-1)
    v2 = jnp.take_along_axis(probs, i2[:, None], axis=-1)[:, 0]

    mask = jnp.zeros((T, E), jnp.int32).at[rows, i1].set((v1 > 0).astype(jnp.int32))
    mask = mask.at[rows, i2].set((v2 > 0).astype(jnp.int32))
    slots = jnp.cumsum(mask, axis=0) - mask
    c1 = jnp.take_along_axis(slots, i1[:, None], axis=-1)[:, 0]
    c2 = jnp.take_along_axis(slots, i2[:, None], axis=-1)[:, 0]
    ok1 = (c1 < cap) & (v1 > 0)
    ok2 = (c2 < cap) & (v2 > 0)
    flat1 = jnp.where(ok1, i1 * cap + c1, dump).astype(jnp.int32)
    flat2 = jnp.where(ok2, i2 * cap + c2, dump).astype(jnp.int32)

    dw = jnp.zeros((n_rows,), jnp.float32).at[flat1].add(jnp.where(ok1, v1, 0.0))
    dw = dw.at[flat2].add(jnp.where(ok2, v2, 0.0))

    ndch = (T // NW) // DCH
    flatd = jnp.stack(
        [flat1.reshape(NW, ndch, DCH), flat2.reshape(NW, ndch, DCH)], axis=2)
    ncch = (T // NW) // CCH
    flatc = jnp.stack(
        [flat1.reshape(NW, ncch, CCH), flat2.reshape(NW, ncch, CCH)],
        axis=2).reshape(NW, ncch * 2 * CCH)

    X = _dispatch(x_flat, flatd, n_rows)
    Yw = _ffn(X, experts_inter, experts_out, dw[:, None], cap, n_rows)
    out = _combine(Yw, flatc, T)
    return out.reshape(b, s, hid)


# speedup vs baseline: 2.2081x; 2.1848x over previous
import functools
import math

import jax
import jax.numpy as jnp
from jax import lax
from jax.experimental import pallas as pl
from jax.experimental.pallas import tpu as pltpu
from jax.experimental.pallas import tpu_sc as plsc

E = 8
HID = 1024
INTER = 2048
NT = 4
TILE_I = INTER // NT
NW = 32


DCH = 32


def _dispatch(x_flat, flatd, n_rows):
    T = x_flat.shape[0]
    tpw = T // NW
    nchunk = tpw // DCH

    @functools.partial(
        pl.kernel,
        out_type=jax.ShapeDtypeStruct((n_rows, HID), jnp.float32),
        mesh=plsc.VectorSubcoreMesh(core_axis_name="c", subcore_axis_name="s"),
        scratch_types=[
            pltpu.VMEM((nchunk, 2, DCH), jnp.int32),
            pltpu.VMEM((DCH, HID), jnp.float32),
            pltpu.VMEM((DCH, HID), jnp.float32),
            pltpu.SemaphoreType.DMA,
            pltpu.SemaphoreType.DMA,
            pltpu.SemaphoreType.DMA,
            pltpu.SemaphoreType.DMA,
        ],
    )
    def k(x_hbm, fd_hbm, X_hbm, idx_v, rows0, rows1, si0, si1, ss0, ss1):
        wid = lax.axis_index("s") * 2 + lax.axis_index("c")
        base = wid * tpw
        bufs = (rows0, rows1)
        isems = (si0, si1)
        ssems = (ss0, ss1)
        pltpu.sync_copy(fd_hbm.at[wid], idx_v)

        def issue_in(j, p):
            @pl.when(j < nchunk)
            def _():
                pltpu.async_copy(
                    x_hbm.at[pl.ds(base + j * DCH, DCH)], bufs[p], isems[p])

        def wait_in(j, p):
            pltpu.make_async_copy(
                x_hbm.at[pl.ds(base + j * DCH, DCH)], bufs[p], isems[p]).wait()

        def scat(j, p):
            pltpu.async_copy(bufs[p], X_hbm.at[idx_v.at[j, 0]], ssems[p])
            pltpu.async_copy(bufs[p], X_hbm.at[idx_v.at[j, 1]], ssems[p])
            pltpu.make_async_copy(bufs[p], X_hbm.at[idx_v.at[j, 0]], ssems[p]).wait()
            pltpu.make_async_copy(bufs[p], X_hbm.at[idx_v.at[j, 1]], ssems[p]).wait()

        issue_in(0, 0)

        def pair(pi, carry):
            i = 2 * pi
            issue_in(i + 1, 1)
            wait_in(i, 0)
            scat(i, 0)
            issue_in(i + 2, 0)
            wait_in(i + 1, 1)
            scat(i + 1, 1)
            return carry

        lax.fori_loop(0, nchunk // 2, pair, 0)

    return k(x_flat, flatd)


CCH = 16


def _combine(Yw, flatc, T):
    tpw = T // NW
    nchunk = tpw // CCH

    @functools.partial(
        pl.kernel,
        out_type=jax.ShapeDtypeStruct((T, HID), jnp.float32),
        mesh=plsc.VectorSubcoreMesh(core_axis_name="c", subcore_axis_name="s"),
        scratch_types=[
            pltpu.VMEM((nchunk * 2 * CCH,), jnp.int32),
            pltpu.VMEM((2 * CCH, HID), jnp.float32),
            pltpu.VMEM((2 * CCH, HID), jnp.float32),
            pltpu.SemaphoreType.DMA,
            pltpu.SemaphoreType.DMA,
            pltpu.SemaphoreType.DMA,
            pltpu.SemaphoreType.DMA,
        ],
    )
    def k(y_hbm, fc_hbm, out_hbm, idx_v, rows0, rows1, sg0, sg1, so0, so1):
        wid = lax.axis_index("s") * 2 + lax.axis_index("c")
        base = wid * tpw
        bufs = (rows0, rows1)
        gsems = (sg0, sg1)
        osems = (so0, so1)
        pltpu.sync_copy(fc_hbm.at[wid], idx_v)

        def issue_gather(j, p):
            @pl.when(j < nchunk)
            def _():
                @pl.when(j >= 2)
                def _():
                    pltpu.make_async_copy(
                        bufs[p].at[pl.ds(0, CCH)],
                        out_hbm.at[pl.ds(base + (j - 2) * CCH, CCH)],
                        osems[p]).wait()

                pltpu.async_copy(
                    y_hbm.at[idx_v.at[pl.ds(j * 2 * CCH, 2 * CCH)]],
                    bufs[p], gsems[p])

        def wait_gather(j, p):
            pltpu.make_async_copy(
                y_hbm.at[idx_v.at[pl.ds(j * 2 * CCH, 2 * CCH)]],
                bufs[p], gsems[p]).wait()

        def process(j, p):
            buf = bufs[p]

            def row_body(r, c):
                for jj in range(HID // 16):
                    sl = pl.ds(jj * 16, 16)
                    buf[r, sl] = buf[r, sl] + buf[CCH + r, sl]
                return c

            lax.fori_loop(0, CCH, row_body, 0)
            pltpu.async_copy(
                buf.at[pl.ds(0, CCH)],
                out_hbm.at[pl.ds(base + j * CCH, CCH)], osems[p])

        issue_gather(0, 0)

        def pair(pi, carry):
            i = 2 * pi
            issue_gather(i + 1, 1)
            wait_gather(i, 0)
            process(i, 0)
            issue_gather(i + 2, 0)
            wait_gather(i + 1, 1)
            process(i + 1, 1)
            return carry

        lax.fori_loop(0, nchunk // 2, pair, 0)
        pltpu.make_async_copy(
            bufs[0].at[pl.ds(0, CCH)],
            out_hbm.at[pl.ds(base + (nchunk - 2) * CCH, CCH)], osems[0]).wait()
        pltpu.make_async_copy(
            bufs[1].at[pl.ds(0, CCH)],
            out_hbm.at[pl.ds(base + (nchunk - 1) * CCH, CCH)], osems[1]).wait()

    return k(Yw, flatc)


def _ffn_body(x_ref, w1_ref, w2_ref, dw_ref, y_ref, acc_ref):
    e = pl.program_id(0)
    nt = pl.program_id(1)

    @pl.when(e < E)
    def _():
        h = jnp.dot(x_ref[...], w1_ref[0], preferred_element_type=jnp.float32)
        h = jnp.maximum(h, 0.0)
        part = jnp.dot(h, w2_ref[0], preferred_element_type=jnp.float32)

        @pl.when(nt == 0)
        def _():
            acc_ref[...] = part

        @pl.when(nt > 0)
        def _():
            acc_ref[...] = acc_ref[...] + part

        @pl.when(nt == NT - 1)
        def _():
            dw = dw_ref[...]
            y_ref[...] = jnp.where(dw > 0.0, acc_ref[...] * dw, 0.0)

    @pl.when((e == E) & (nt == NT - 1))
    def _():
        y_ref[...] = jnp.zeros_like(y_ref)


def _ffn(x_disp, experts_inter, experts_out, dw, cap, n_rows):
    return pl.pallas_call(
        _ffn_body,
        grid=(E + 1, NT),
        in_specs=[
            pl.BlockSpec((cap, HID), lambda e, n: (e, 0)),
            pl.BlockSpec((1, HID, TILE_I), lambda e, n: (jnp.minimum(e, E - 1), 0, n)),
            pl.BlockSpec((1, TILE_I, HID), lambda e, n: (jnp.minimum(e, E - 1), n, 0)),
            pl.BlockSpec((cap, 1), lambda e, n: (e, 0)),
        ],
        out_specs=pl.BlockSpec((cap, HID), lambda e, n: (e, 0)),
        out_shape=jax.ShapeDtypeStruct((n_rows, HID), jnp.float32),
        scratch_shapes=[pltpu.VMEM((cap, HID), jnp.float32)],
        compiler_params=pltpu.CompilerParams(
            dimension_semantics=("arbitrary", "arbitrary"),
        ),
    )(x_disp, experts_inter, experts_out, dw)


def kernel(x, experts_inter, experts_out, router_w, router_b):
    b, s, hid = x.shape
    T = b * s
    cap = math.ceil(T / E * 1.0)
    n_rows = (E + 1) * cap
    dump = E * cap
    x_flat = x.reshape(T, hid)

    logits = x_flat @ router_w.T + router_b
    probs = jax.nn.softmax(logits, axis=-1)
    rows = jnp.arange(T)
    i1 = jnp.argmax(probs, axis=-1)
    v1 = jnp.take_along_axis(probs, i1[:, None], axis=-1)[:, 0]
    masked = probs.at[rows, i1].set(-jnp.inf)
    i2 = jnp.argmax(masked, axis=-1)
    v2 = jnp.take_along_axis(probs, i2[:, None], axis=-1)[:, 0]

    mask = jnp.zeros((T, E), jnp.int32).at[rows, i1].set((v1 > 0).astype(jnp.int32))
    mask = mask.at[rows, i2].set((v2 > 0).astype(jnp.int32))
    slots = jnp.cumsum(mask, axis=0) - mask
    c1 = jnp.take_along_axis(slots, i1[:, None], axis=-1)[:, 0]
    c2 = jnp.take_along_axis(slots, i2[:, None], axis=-1)[:, 0]
    ok1 = (c1 < cap) & (v1 > 0)
    ok2 = (c2 < cap) & (v2 > 0)
    spread = dump + (rows % cap)
    flat1 = jnp.where(ok1, i1 * cap + c1, spread).astype(jnp.int32)
    flat2 = jnp.where(ok2, i2 * cap + c2, spread).astype(jnp.int32)

    dw = jnp.zeros((n_rows,), jnp.float32).at[flat1].add(jnp.where(ok1, v1, 0.0))
    dw = dw.at[flat2].add(jnp.where(ok2, v2, 0.0))

    ndch = (T // NW) // DCH
    flatd = jnp.stack(
        [flat1.reshape(NW, ndch, DCH), flat2.reshape(NW, ndch, DCH)], axis=2)
    ncch = (T // NW) // CCH
    flatc = jnp.stack(
        [flat1.reshape(NW, ncch, CCH), flat2.reshape(NW, ncch, CCH)],
        axis=2).reshape(NW, ncch * 2 * CCH)

    X = _dispatch(x_flat, flatd, n_rows)
    Yw = _ffn(X, experts_inter, experts_out, dw[:, None], cap, n_rows)
    out = _combine(Yw, flatc, T)
    return out.reshape(b, s, hid)


# speedup vs baseline: 3.7360x; 1.6920x over previous
import functools
import math

import jax
import jax.numpy as jnp
from jax import lax
from jax.experimental import pallas as pl
from jax.experimental.pallas import tpu as pltpu
from jax.experimental.pallas import tpu_sc as plsc

E = 8
HID = 1024
INTER = 2048
NT = 4
TILE_I = INTER // NT
NW = 32


RB = 256


def _routing_body(cap, nblk, x_ref, w_ref, b_ref, f1_ref, f2_ref, w1_ref,
                  w2_ref, offs_ref):
    blk = pl.program_id(0)
    dump = E * cap

    @pl.when(blk == 0)
    def _():
        offs_ref[...] = jnp.zeros_like(offs_ref)

    logits = jax.lax.dot_general(
        x_ref[...], w_ref[...], (((1,), (1,)), ((), ())),
        preferred_element_type=jnp.float32) + b_ref[...]
    m = jnp.max(logits, axis=-1, keepdims=True)
    ex = jnp.exp(logits - m)
    probs = ex / jnp.sum(ex, axis=-1, keepdims=True)

    e_ids = jax.lax.broadcasted_iota(jnp.int32, (RB, E), 1)
    v1 = jnp.max(probs, axis=-1, keepdims=True)
    i1 = jnp.min(jnp.where(probs == v1, e_ids, E), axis=-1, keepdims=True)
    is1 = e_ids == i1
    masked = jnp.where(is1, -jnp.inf, probs)
    v2 = jnp.max(masked, axis=-1, keepdims=True)
    i2 = jnp.min(jnp.where((masked == v2) & ~is1, e_ids, E),
                 axis=-1, keepdims=True)
    is2 = e_ids == i2

    mask = (is1 & (v1 > 0)) | (is2 & (v2 > 0))
    mask_f = mask.astype(jnp.float32)
    r_iota = jax.lax.broadcasted_iota(jnp.int32, (RB, RB), 0)
    c_iota = jax.lax.broadcasted_iota(jnp.int32, (RB, RB), 1)
    tril = jnp.where(r_iota > c_iota, 1.0, 0.0)
    slots = jnp.dot(tril, mask_f, preferred_element_type=jnp.float32)
    slots = slots + offs_ref[...]
    offs_ref[...] = offs_ref[...] + jnp.sum(mask_f, axis=0, keepdims=True)

    c1 = jnp.sum(jnp.where(is1, slots, 0.0), axis=-1).astype(jnp.int32)
    c2 = jnp.sum(jnp.where(is2, slots, 0.0), axis=-1).astype(jnp.int32)
    v1f = v1[:, 0]
    v2f = v2[:, 0]
    i1f = i1[:, 0]
    i2f = i2[:, 0]
    ok1 = (c1 < cap) & (v1f > 0)
    ok2 = (c2 < cap) & (v2f > 0)
    t_ids = blk * RB + jax.lax.broadcasted_iota(jnp.int32, (RB,), 0)
    spread = dump + (t_ids % cap)
    f1_ref[0, 0, :] = jnp.where(ok1, i1f * cap + c1, spread)
    f2_ref[0, 0, :] = jnp.where(ok2, i2f * cap + c2, spread)
    w1_ref[0, 0, :] = jnp.where(ok1, v1f, 0.0)
    w2_ref[0, 0, :] = jnp.where(ok2, v2f, 0.0)


def _routing(x_flat, router_w, router_b, cap):
    T = x_flat.shape[0]
    nblk = T // RB
    out3 = lambda dt: jax.ShapeDtypeStruct((nblk, 1, RB), dt)
    spec3 = pl.BlockSpec((1, 1, RB), lambda b: (b, 0, 0))
    f1, f2, w1, w2 = pl.pallas_call(
        functools.partial(_routing_body, cap, nblk),
        grid=(nblk,),
        in_specs=[
            pl.BlockSpec((RB, HID), lambda b: (b, 0)),
            pl.BlockSpec((E, HID), lambda b: (0, 0)),
            pl.BlockSpec((1, E), lambda b: (0, 0)),
        ],
        out_specs=[spec3, spec3, spec3, spec3],
        out_shape=[out3(jnp.int32), out3(jnp.int32),
                   out3(jnp.float32), out3(jnp.float32)],
        scratch_shapes=[pltpu.VMEM((1, E), jnp.float32)],
        compiler_params=pltpu.CompilerParams(
            dimension_semantics=("arbitrary",),
        ),
    )(x_flat, router_w, router_b.reshape(1, E))
    return (f1.reshape(T), f2.reshape(T), w1.reshape(T), w2.reshape(T))


DCH = 32


def _dispatch(x_flat, flatd, n_rows):
    T = x_flat.shape[0]
    tpw = T // NW
    nchunk = tpw // DCH

    @functools.partial(
        pl.kernel,
        out_type=jax.ShapeDtypeStruct((n_rows, HID), jnp.float32),
        mesh=plsc.VectorSubcoreMesh(core_axis_name="c", subcore_axis_name="s"),
        scratch_types=[
            pltpu.VMEM((nchunk, 2, DCH), jnp.int32),
            pltpu.VMEM((DCH, HID), jnp.float32),
            pltpu.VMEM((DCH, HID), jnp.float32),
            pltpu.SemaphoreType.DMA,
            pltpu.SemaphoreType.DMA,
            pltpu.SemaphoreType.DMA,
            pltpu.SemaphoreType.DMA,
        ],
    )
    def k(x_hbm, fd_hbm, X_hbm, idx_v, rows0, rows1, si0, si1, ss0, ss1):
        wid = lax.axis_index("s") * 2 + lax.axis_index("c")
        base = wid * tpw
        bufs = (rows0, rows1)
        isems = (si0, si1)
        ssems = (ss0, ss1)
        pltpu.sync_copy(fd_hbm.at[wid], idx_v)

        def issue_in(j, p):
            @pl.when(j < nchunk)
            def _():
                pltpu.async_copy(
                    x_hbm.at[pl.ds(base + j * DCH, DCH)], bufs[p], isems[p])

        def wait_in(j, p):
            pltpu.make_async_copy(
                x_hbm.at[pl.ds(base + j * DCH, DCH)], bufs[p], isems[p]).wait()

        def scat(j, p):
            pltpu.async_copy(bufs[p], X_hbm.at[idx_v.at[j, 0]], ssems[p])
            pltpu.async_copy(bufs[p], X_hbm.at[idx_v.at[j, 1]], ssems[p])
            pltpu.make_async_copy(bufs[p], X_hbm.at[idx_v.at[j, 0]], ssems[p]).wait()
            pltpu.make_async_copy(bufs[p], X_hbm.at[idx_v.at[j, 1]], ssems[p]).wait()

        issue_in(0, 0)

        def pair(pi, carry):
            i = 2 * pi
            issue_in(i + 1, 1)
            wait_in(i, 0)
            scat(i, 0)
            issue_in(i + 2, 0)
            wait_in(i + 1, 1)
            scat(i + 1, 1)
            return carry

        lax.fori_loop(0, nchunk // 2, pair, 0)

    return k(x_flat, flatd)


CCH = 16


def _combine(Yw, flatc, wc, T):
    tpw = T // NW
    nchunk = tpw // CCH

    @functools.partial(
        pl.kernel,
        out_type=jax.ShapeDtypeStruct((T, HID), jnp.float32),
        mesh=plsc.VectorSubcoreMesh(core_axis_name="c", subcore_axis_name="s"),
        scratch_types=[
            pltpu.VMEM((nchunk * 2 * CCH,), jnp.int32),
            pltpu.VMEM((nchunk, 2, CCH), jnp.float32),
            pltpu.VMEM((2 * CCH, HID), jnp.float32),
            pltpu.VMEM((2 * CCH, HID), jnp.float32),
            pltpu.SemaphoreType.DMA,
            pltpu.SemaphoreType.DMA,
            pltpu.SemaphoreType.DMA,
            pltpu.SemaphoreType.DMA,
        ],
    )
    def k(y_hbm, fc_hbm, wc_hbm, out_hbm, idx_v, wv, rows0, rows1,
          sg0, sg1, so0, so1):
        wid = lax.axis_index("s") * 2 + lax.axis_index("c")
        base = wid * tpw
        bufs = (rows0, rows1)
        gsems = (sg0, sg1)
        osems = (so0, so1)
        pltpu.sync_copy(fc_hbm.at[wid], idx_v)
        pltpu.sync_copy(wc_hbm.at[wid], wv)

        def issue_gather(j, p):
            @pl.when(j < nchunk)
            def _():
                @pl.when(j >= 2)
                def _():
                    pltpu.make_async_copy(
                        bufs[p].at[pl.ds(0, CCH)],
                        out_hbm.at[pl.ds(base + (j - 2) * CCH, CCH)],
                        osems[p]).wait()

                pltpu.async_copy(
                    y_hbm.at[idx_v.at[pl.ds(j * 2 * CCH, 2 * CCH)]],
                    bufs[p], gsems[p])

        def wait_gather(j, p):
            pltpu.make_async_copy(
                y_hbm.at[idx_v.at[pl.ds(j * 2 * CCH, 2 * CCH)]],
                bufs[p], gsems[p]).wait()

        def process(j, p):
            buf = bufs[p]
            w1v = wv[j, 0, :]
            w2v = wv[j, 1, :]

            gdims = lax.GatherDimensionNumbers(
                offset_dims=(), collapsed_slice_dims=(0,), start_index_map=(0,))

            def row_body(r, c):
                rfull = jnp.full((16, 1), r, jnp.int32)
                bw1 = lax.gather(
                    w1v, rfull, gdims, (1,),
                    mode=lax.GatherScatterMode.PROMISE_IN_BOUNDS)
                bw2 = lax.gather(
                    w2v, rfull, gdims, (1,),
                    mode=lax.GatherScatterMode.PROMISE_IN_BOUNDS)
                for jj in range(HID // 16):
                    sl = pl.ds(jj * 16, 16)
                    buf[r, sl] = buf[r, sl] * bw1 + buf[CCH + r, sl] * bw2
                return c

            lax.fori_loop(0, CCH, row_body, 0)
            pltpu.async_copy(
                buf.at[pl.ds(0, CCH)],
                out_hbm.at[pl.ds(base + j * CCH, CCH)], osems[p])

        issue_gather(0, 0)

        def pair(pi, carry):
            i = 2 * pi
            issue_gather(i + 1, 1)
            wait_gather(i, 0)
            process(i, 0)
            issue_gather(i + 2, 0)
            wait_gather(i + 1, 1)
            process(i + 1, 1)
            return carry

        lax.fori_loop(0, nchunk // 2, pair, 0)
        pltpu.make_async_copy(
            bufs[0].at[pl.ds(0, CCH)],
            out_hbm.at[pl.ds(base + (nchunk - 2) * CCH, CCH)], osems[0]).wait()
        pltpu.make_async_copy(
            bufs[1].at[pl.ds(0, CCH)],
            out_hbm.at[pl.ds(base + (nchunk - 1) * CCH, CCH)], osems[1]).wait()

    return k(Yw, flatc, wc)


def _ffn_body(x_ref, w1_ref, w2_ref, y_ref, acc_ref):
    e = pl.program_id(0)
    nt = pl.program_id(1)

    @pl.when(e < E)
    def _():
        h = jnp.dot(x_ref[...], w1_ref[0], preferred_element_type=jnp.float32)
        h = jnp.maximum(h, 0.0)
        part = jnp.dot(h, w2_ref[0], preferred_element_type=jnp.float32)

        @pl.when(nt == 0)
        def _():
            acc_ref[...] = part

        @pl.when(nt > 0)
        def _():
            acc_ref[...] = acc_ref[...] + part

        @pl.when(nt == NT - 1)
        def _():
            y_ref[...] = acc_ref[...]

    @pl.when((e == E) & (nt == NT - 1))
    def _():
        y_ref[...] = jnp.zeros_like(y_ref)


def _ffn(x_disp, experts_inter, experts_out, cap, n_rows):
    return pl.pallas_call(
        _ffn_body,
        grid=(E + 1, NT),
        in_specs=[
            pl.BlockSpec((cap, HID), lambda e, n: (e, 0)),
            pl.BlockSpec((1, HID, TILE_I), lambda e, n: (jnp.minimum(e, E - 1), 0, n)),
            pl.BlockSpec((1, TILE_I, HID), lambda e, n: (jnp.minimum(e, E - 1), n, 0)),
        ],
        out_specs=pl.BlockSpec((cap, HID), lambda e, n: (e, 0)),
        out_shape=jax.ShapeDtypeStruct((n_rows, HID), jnp.float32),
        scratch_shapes=[pltpu.VMEM((cap, HID), jnp.float32)],
        compiler_params=pltpu.CompilerParams(
            dimension_semantics=("arbitrary", "arbitrary"),
        ),
    )(x_disp, experts_inter, experts_out)


def kernel(x, experts_inter, experts_out, router_w, router_b):
    b, s, hid = x.shape
    T = b * s
    cap = math.ceil(T / E * 1.0)
    n_rows = (E + 1) * cap
    x_flat = x.reshape(T, hid)

    flat1, flat2, w1, w2 = _routing(x_flat, router_w, router_b, cap)

    ndch = (T // NW) // DCH
    flatd = jnp.stack(
        [flat1.reshape(NW, ndch, DCH), flat2.reshape(NW, ndch, DCH)], axis=2)
    ncch = (T // NW) // CCH
    flatc = jnp.stack(
        [flat1.reshape(NW, ncch, CCH), flat2.reshape(NW, ncch, CCH)],
        axis=2).reshape(NW, ncch * 2 * CCH)
    wc = jnp.stack(
        [w1.reshape(NW, ncch, CCH), w2.reshape(NW, ncch, CCH)], axis=2)

    X = _dispatch(x_flat, flatd, n_rows)
    Y = _ffn(X, experts_inter, experts_out, cap, n_rows)
    out = _combine(Y, flatc, wc, T)
    return out.reshape(b, s, hid)
